# baseline (speedup 1.0000x reference)
import functools

import jax
import jax.numpy as jnp
from jax import lax
from jax.experimental import pallas as pl
from jax.experimental.pallas import tpu as pltpu
from jax.experimental.pallas import tpu_sc as plsc

N = 10000
E = 320000
D = 128

NC = 2
NS = 16
HD = D // NC
PAD_N = 10240
STRIPE = E // NS
CHUNK = 80
TRIPS = STRIPE // CHUNK
ROWS_PER_TILE = PAD_N // NS
ZROWS = 512


def _spmm_body(x_hbm, src_hbm, dst_hbm, w_hbm, binit_hbm, out_hbm,
               rows0_v, rows1_v, rows2_v,
               src0_v, src1_v, src2_v,
               dst0_v, dst1_v, dst2_v,
               w0_v, w1_v, w2_v,
               adj0_v, adj1_v, adj2_v,
               dstst0_v, dstst1_v, dstst2_v, accum,
               gsem0, gsem1, gsem2, ssem0, ssem1, ssem2,
               esem0, esem1, esem2):
    bufs = (rows0_v, rows1_v, rows2_v)
    srcs = (src0_v, src1_v, src2_v)
    dsts = (dst0_v, dst1_v, dst2_v)
    ws = (w0_v, w1_v, w2_v)
    adjs = (adj0_v, adj1_v, adj2_v)
    dststs = (dstst0_v, dstst1_v, dstst2_v)
    gsems = (gsem0, gsem1, gsem2)
    ssems = (ssem0, ssem1, ssem2)
    esems = (esem0, esem1, esem2)
    c = lax.axis_index("c")
    s = lax.axis_index("s")

    row0 = s * ROWS_PER_TILE
    done = 0
    while done < ROWS_PER_TILE:
        step = min(ZROWS, ROWS_PER_TILE - done)
        pltpu.sync_copy(binit_hbm.at[c, pl.ds(0, step)],
                        accum.at[pl.ds(row0 + done, step)])
        done += step

    plsc.subcore_barrier()

    xoff_v = jnp.broadcast_to(c * N, (16,))

    def start_edges(j, b):
        pltpu.async_copy(src_hbm.at[s, j], srcs[b], esems[b])
        pltpu.async_copy(dst_hbm.at[s, j], dsts[b], esems[b])
        pltpu.async_copy(w_hbm.at[s, j], ws[b], esems[b])

    def wait_edges(b):
        pltpu.make_async_copy(src_hbm.at[0, 0], srcs[b], esems[b]).wait()
        pltpu.make_async_copy(dst_hbm.at[0, 0], dsts[b], esems[b]).wait()
        pltpu.make_async_copy(w_hbm.at[0, 0], ws[b], esems[b]).wait()

    def start_gather(b):
        for g in range(CHUNK // 16):
            adjs[b][pl.ds(g * 16, 16)] = (
                srcs[b][pl.ds(g * 16, 16)] + xoff_v)
        pltpu.async_copy(x_hbm.at[adjs[b]], bufs[b], gsems[b])

    def wait_gather(b):
        pltpu.make_async_copy(x_hbm.at[pl.ds(0, CHUNK)], bufs[b],
                              gsems[b]).wait()

    def wait_scatter(b):
        pltpu.make_async_copy(bufs[b], accum.at[pl.ds(0, CHUNK)],
                              ssems[b]).wait()

    def handler(b, j):
        buf, dstst = bufs[b], dststs[b]
        wait_gather(b)

        def group(g, _):
            w16 = ws[b][pl.ds(g * 16, 16)]
            dstst[0, pl.ds(g * 16, 16)] = dsts[b][pl.ds(g * 16, 16)]
            for l in range(16):
                wvec = jnp.broadcast_to(w16[l], (16,))
                e = g * 16 + l
                for q in range(HD // 16):
                    buf[e, pl.ds(q * 16, 16)] = (
                        buf[e, pl.ds(q * 16, 16)] * wvec)
            return 0
        lax.fori_loop(0, CHUNK // 16, group, 0)
        pltpu.async_copy(buf, accum.at[dstst.at[0]], ssems[b], add=True)

        b2 = (b + 2) % 3

        @pl.when(j >= 1)
        def _():
            wait_scatter(b2)

        @pl.when(j + 2 < TRIPS)
        def _():
            wait_edges(b2)
            start_gather(b2)

        @pl.when(j + 3 < TRIPS)
        def _():
            start_edges(j + 3, b)

    for b in range(3):
        start_edges(b, b)
    for b in range(2):
        wait_edges(b)
        start_gather(b)

    def tri_body(k, _):
        for b in range(3):
            handler(b, 3 * k + b)
        return 0
    lax.fori_loop(0, TRIPS // 3, tri_body, 0)
    for t in range(TRIPS - (TRIPS // 3) * 3):
        handler(t, jnp.int32((TRIPS // 3) * 3 + t))
    wait_scatter((TRIPS - 1) % 3)

    plsc.subcore_barrier()
    pltpu.sync_copy(accum.at[pl.ds(row0, ROWS_PER_TILE)],
                    out_hbm.at[c, pl.ds(row0, ROWS_PER_TILE)])


_spmm = functools.partial(
    pl.kernel,
    out_type=jax.ShapeDtypeStruct((NC, PAD_N, HD), jnp.float32),
    mesh=plsc.VectorSubcoreMesh(core_axis_name="c", subcore_axis_name="s"),
    compiler_params=pltpu.CompilerParams(use_tc_tiling_on_sc=False),
    scratch_types=(
        [pltpu.VMEM((CHUNK, HD), jnp.float32)] * 3
        + [pltpu.VMEM((CHUNK,), jnp.int32)] * 3
        + [pltpu.VMEM((CHUNK,), jnp.int32)] * 3
        + [pltpu.VMEM((CHUNK,), jnp.float32)] * 3
        + [pltpu.VMEM((CHUNK,), jnp.int32)] * 3
        + [pltpu.VMEM((1, CHUNK), jnp.int32)] * 3
        + [pltpu.VMEM_SHARED((PAD_N, HD), jnp.float32)]
        + [pltpu.SemaphoreType.DMA] * 9),
)(_spmm_body)


BLK = N // 10


def _mm_body(x_ref, w_ref, o_ref):
    o_ref[0] = jnp.dot(x_ref[...], w_ref[0],
                       preferred_element_type=jnp.float32)


def _mm_split(x, w):
    return pl.pallas_call(
        _mm_body,
        grid=(10, NC),
        in_specs=[
            pl.BlockSpec((BLK, D), lambda i, j: (i, 0)),
            pl.BlockSpec((1, D, HD), lambda i, j: (j, 0, 0)),
        ],
        out_specs=pl.BlockSpec((1, BLK, HD), lambda i, j: (j, i, 0)),
        out_shape=jax.ShapeDtypeStruct((NC, N, HD), jnp.float32),
    )(x, w)


def _relu_mm_body(h_ref, w_ref, o_ref):
    h = jnp.concatenate([h_ref[0], h_ref[1]], axis=-1)
    o_ref[0] = jnp.dot(jax.nn.relu(h), w_ref[0],
                       preferred_element_type=jnp.float32)


def _relu_mm_split(h, w):
    return pl.pallas_call(
        _relu_mm_body,
        grid=(10, NC),
        in_specs=[
            pl.BlockSpec((NC, BLK, HD), lambda i, j: (0, i, 0)),
            pl.BlockSpec((1, D, HD), lambda i, j: (j, 0, 0)),
        ],
        out_specs=pl.BlockSpec((1, BLK, HD), lambda i, j: (j, i, 0)),
        out_shape=jax.ShapeDtypeStruct((NC, N, HD), jnp.float32),
    )(h, w)


def kernel(embeddings, edge_index, edge_weight, W1, b1, W2, b2):
    src = edge_index[0].reshape(NS, TRIPS, CHUNK)
    dst = edge_index[1].reshape(NS, TRIPS, CHUNK)
    ww = edge_weight.reshape(NS, TRIPS, CHUNK)
    binit1 = jnp.broadcast_to(b1.reshape(NC, 1, HD), (NC, ZROWS, HD))
    binit2 = jnp.broadcast_to(b2.reshape(NC, 1, HD), (NC, ZROWS, HD))

    w1h = jnp.stack([W1[:, :HD], W1[:, HD:]])
    w2h = jnp.stack([W2[:, :HD], W2[:, HD:]])
    s1 = _mm_split(embeddings, w1h)
    h1 = _spmm(s1.reshape(NC * N, HD),
               src, dst, ww, binit1)
    s2 = _relu_mm_split(h1, w2h)
    out = _spmm(s2.reshape(NC * N, HD),
                src, dst, ww, binit2)
    return jnp.concatenate([out[0, :N], out[1, :N]], axis=1)


# speedup vs baseline: 1.6741x; 1.6741x over previous
import functools

import jax
import jax.numpy as jnp
from jax import lax
from jax.experimental import pallas as pl
from jax.experimental.pallas import tpu as pltpu
from jax.experimental.pallas import tpu_sc as plsc

N = 10000
E = 320000
D = 128

NC = 2
NS = 16
NW = NC * NS
PAD_N = 10240
HALF = PAD_N // NC
TRASH = HALF
CHUNK = 80
ZROWS = 512


PART_N = 10112


def _make_spmm_body(partials):
    stripe = E // NW if partials else E // NS
    trips = stripe // CHUNK
    acc_rows = PART_N if partials else HALF
    rows_per_tile = acc_rows // NS

    def body(x_hbm, src_hbm, dst_hbm, w_hbm, binit_hbm, out_hbm,
             rows0_v, rows1_v, rows2_v,
             src0_v, src1_v, src2_v,
             dst0_v, dst1_v, dst2_v,
             w0_v, w1_v, w2_v,
             dstst0_v, dstst1_v, dstst2_v, accum,
             gsem0, gsem1, gsem2, ssem0, ssem1, ssem2,
             esem0, esem1, esem2):
        bufs = (rows0_v, rows1_v, rows2_v)
        srcs = (src0_v, src1_v, src2_v)
        dsts = (dst0_v, dst1_v, dst2_v)
        ws = (w0_v, w1_v, w2_v)
        dststs = (dstst0_v, dstst1_v, dstst2_v)
        gsems = (gsem0, gsem1, gsem2)
        ssems = (ssem0, ssem1, ssem2)
        esems = (esem0, esem1, esem2)
        c = lax.axis_index("c")
        s = lax.axis_index("s")
        wid = c * NS + s
        esl = wid if partials else s

        row0 = s * rows_per_tile
        done = 0
        while done < rows_per_tile:
            step = min(ZROWS, rows_per_tile - done)
            pltpu.sync_copy(binit_hbm.at[c, pl.ds(0, step)],
                            accum.at[pl.ds(row0 + done, step)])
            done += step

        plsc.subcore_barrier()

        lo_v = jnp.broadcast_to(c * HALF, (16,))
        half_v = jnp.full((16,), HALF, jnp.int32)
        trash_v = jnp.full((16,), TRASH, jnp.int32)
        zero_v = jnp.zeros((16,), jnp.int32)

        def start_edges(j, b):
            pltpu.async_copy(src_hbm.at[esl, j], srcs[b], esems[b])
            pltpu.async_copy(dst_hbm.at[esl, j], dsts[b], esems[b])
            pltpu.async_copy(w_hbm.at[esl, j], ws[b], esems[b])

        def wait_edges(b):
            pltpu.make_async_copy(src_hbm.at[0, 0], srcs[b], esems[b]).wait()
            pltpu.make_async_copy(dst_hbm.at[0, 0], dsts[b], esems[b]).wait()
            pltpu.make_async_copy(w_hbm.at[0, 0], ws[b], esems[b]).wait()

        def start_gather(b):
            pltpu.async_copy(x_hbm.at[srcs[b]], bufs[b], gsems[b])

        def wait_gather(b):
            pltpu.make_async_copy(x_hbm.at[pl.ds(0, CHUNK)], bufs[b],
                                  gsems[b]).wait()

        def wait_scatter(b):
            pltpu.make_async_copy(bufs[b], accum.at[pl.ds(0, CHUNK)],
                                  ssems[b]).wait()

        def handler(b, j):
            buf, dstst = bufs[b], dststs[b]
            wait_gather(b)

            def group(g, _):
                w16 = ws[b][pl.ds(g * 16, 16)]
                d16 = dsts[b][pl.ds(g * 16, 16)]
                if not partials:
                    d16 = d16 - lo_v
                    d16 = jnp.where(d16 >= zero_v,
                                    jnp.where(d16 < half_v, d16, trash_v),
                                    trash_v)
                dstst[0, pl.ds(g * 16, 16)] = d16
                for l in range(16):
                    wvec = jnp.broadcast_to(w16[l], (16,))
                    e = g * 16 + l
                    for q in range(D // 16):
                        buf[e, pl.ds(q * 16, 16)] = (
                            buf[e, pl.ds(q * 16, 16)] * wvec)
                return 0
            lax.fori_loop(0, CHUNK // 16, group, 0)
            pltpu.async_copy(buf, accum.at[dstst.at[0]], ssems[b], add=True)

            b2 = (b + 2) % 3

            @pl.when(j >= 1)
            def _():
                wait_scatter(b2)

            @pl.when(j + 2 < trips)
            def _():
                wait_edges(b2)
                start_gather(b2)

            @pl.when(j + 3 < trips)
            def _():
                start_edges(j + 3, b)

        for b in range(3):
            start_edges(b, b)
        for b in range(2):
            wait_edges(b)
            start_gather(b)

        def tri_body(k, _):
            for b in range(3):
                handler(b, 3 * k + b)
            return 0
        lax.fori_loop(0, trips // 3, tri_body, 0)
        for t in range(trips - (trips // 3) * 3):
            handler(t, jnp.int32((trips // 3) * 3 + t))
        wait_scatter((trips - 1) % 3)

        plsc.subcore_barrier()
        if partials:
            pltpu.sync_copy(accum.at[pl.ds(row0, rows_per_tile)],
                            out_hbm.at[c, pl.ds(row0, rows_per_tile)])
        else:
            pltpu.sync_copy(accum.at[pl.ds(row0, rows_per_tile)],
                            out_hbm.at[pl.ds(c * HALF + row0, rows_per_tile)])

    return body, stripe, acc_rows


def _make_spmm(partials):
    body, stripe, acc_rows = _make_spmm_body(partials)
    out_shape = ((NC, PART_N, D) if partials else (PAD_N, D))
    return functools.partial(
        pl.kernel,
        out_type=jax.ShapeDtypeStruct(out_shape, jnp.float32),
        mesh=plsc.VectorSubcoreMesh(core_axis_name="c", subcore_axis_name="s"),
        scratch_types=(
            [pltpu.VMEM((CHUNK, D), jnp.float32)] * 3
            + [pltpu.VMEM((CHUNK,), jnp.int32)] * 3
            + [pltpu.VMEM((CHUNK,), jnp.int32)] * 3
            + [pltpu.VMEM((CHUNK,), jnp.float32)] * 3
            + [pltpu.VMEM((1, CHUNK), jnp.int32)] * 3
            + [pltpu.VMEM_SHARED((acc_rows + 8, D), jnp.float32)]
            + [pltpu.SemaphoreType.DMA] * 9),
    )(body)


_spmm_partials = _make_spmm(True)
_spmm_split = _make_spmm(False)


def _mm_body(x_ref, w_ref, o_ref):
    o_ref[...] = jnp.dot(x_ref[...], w_ref[...],
                         preferred_element_type=jnp.float32)


def _mm(x, w):
    blk = N // 10
    return pl.pallas_call(
        _mm_body,
        grid=(10,),
        in_specs=[
            pl.BlockSpec((blk, D), lambda i: (i, 0)),
            pl.BlockSpec((D, D), lambda i: (0, 0)),
        ],
        out_specs=pl.BlockSpec((blk, D), lambda i: (i, 0)),
        out_shape=jax.ShapeDtypeStruct((N, D), jnp.float32),
    )(x, w)


def _fused_mm_body(p_ref, w_ref, o_ref):
    h = jax.nn.relu(p_ref[0] + p_ref[1])
    o_ref[...] = jnp.dot(h, w_ref[...], preferred_element_type=jnp.float32)


def _fused_mm(p, w):
    blk = N // 10
    return pl.pallas_call(
        _fused_mm_body,
        grid=(10,),
        in_specs=[
            pl.BlockSpec((NC, blk, D), lambda i: (0, i, 0)),
            pl.BlockSpec((D, D), lambda i: (0, 0)),
        ],
        out_specs=pl.BlockSpec((blk, D), lambda i: (i, 0)),
        out_shape=jax.ShapeDtypeStruct((N, D), jnp.float32),
    )(p, w)


def kernel(embeddings, edge_index, edge_weight, W1, b1, W2, b2):
    src1 = edge_index[0].reshape(NW, (E // NW) // CHUNK, CHUNK)
    dst1 = edge_index[1].reshape(NW, (E // NW) // CHUNK, CHUNK)
    ww1 = edge_weight.reshape(NW, (E // NW) // CHUNK, CHUNK)
    src2 = edge_index[0].reshape(NS, (E // NS) // CHUNK, CHUNK)
    dst2 = edge_index[1].reshape(NS, (E // NS) // CHUNK, CHUNK)
    ww2 = edge_weight.reshape(NS, (E // NS) // CHUNK, CHUNK)
    zb = jnp.zeros((ZROWS, D), jnp.float32)
    binit1 = jnp.stack([jnp.broadcast_to(b1, (ZROWS, D)), zb])
    binit2 = jnp.broadcast_to(b2, (NC, ZROWS, D))

    s1 = _mm(embeddings, W1)
    p = _spmm_partials(s1, src1, dst1, ww1, binit1)
    s2 = _fused_mm(p, W2)
    out = _spmm_split(s2, src2, dst2, ww2, binit2)
    return out[:N]
